# baseline (speedup 1.0000x reference)
import functools

import jax
import jax.numpy as jnp
import numpy as np
from jax.experimental import pallas as pl


def _conv2d(x, w, b, stride, pad):
    out = jax.lax.conv_general_dilated(
        x, w, window_strides=(stride, stride), padding=((pad, pad), (pad, pad)),
        dimension_numbers=('NCHW', 'OIHW', 'NCHW'))
    return out + b[None, :, None, None]


def _deconv2d(x, w, b, stride, pad):
    k = w.shape[2]
    w_t = jnp.transpose(jnp.flip(w, axis=(2, 3)), (1, 0, 2, 3))
    q = k - 1 - pad
    out = jax.lax.conv_general_dilated(
        x, w_t, window_strides=(1, 1), padding=((q, q), (q, q)),
        lhs_dilation=(stride, stride), dimension_numbers=('NCHW', 'OIHW', 'NCHW'))
    return out + b[None, :, None, None]


def _resize_bilinear_align_corners(x, out_h, out_w):
    H = x.shape[2]
    W = x.shape[3]
    ys = jnp.linspace(0.0, H - 1.0, out_h)
    y0 = jnp.floor(ys).astype(jnp.int32)
    y1 = jnp.minimum(y0 + 1, H - 1)
    wy = (ys - y0.astype(ys.dtype)).astype(x.dtype)
    xs = jnp.linspace(0.0, W - 1.0, out_w)
    x0 = jnp.floor(xs).astype(jnp.int32)
    x1 = jnp.minimum(x0 + 1, W - 1)
    wx = (xs - x0.astype(xs.dtype)).astype(x.dtype)
    top = x[:, :, y0, :] * (1.0 - wy)[None, None, :, None] + x[:, :, y1, :] * wy[None, None, :, None]
    out = top[:, :, :, x0] * (1.0 - wx)[None, None, None, :] + top[:, :, :, x1] * wx[None, None, None, :]
    return out


_RB = 1568


def _vq_body(flat_ref, emb_ref, idx_ref, quant_ref):
    f = flat_ref[...]
    e = emb_ref[...]
    prod = jax.lax.dot_general(f, e, (((1,), (1,)), ((), ())),
                               preferred_element_type=jnp.float32)
    e2row = jax.lax.dot_general(
        jnp.ones((1, 32), jnp.float32), e * e, (((1,), (1,)), ((), ())),
        precision=jax.lax.Precision.HIGHEST,
        preferred_element_type=jnp.float32)
    scores = e2row - 2.0 * prod
    m = jnp.min(scores, axis=1, keepdims=True)
    iota = jax.lax.broadcasted_iota(jnp.int32, scores.shape, 1)
    idxv = jnp.min(jnp.where(scores == m, iota, 512), axis=1,
                   keepdims=True)
    idx_ref[...] = idxv
    onehot = (iota == idxv).astype(jnp.float32)
    quant_ref[...] = jax.lax.dot_general(
        onehot, e, (((1,), (0,)), ((), ())),
        precision=jax.lax.Precision.HIGHEST,
        preferred_element_type=jnp.float32)


def _vq(flat, emb):
    n = flat.shape[0]
    grid = n // _RB
    idx3, quant = pl.pallas_call(
        _vq_body,
        grid=(grid,),
        in_specs=[
            pl.BlockSpec((_RB, 32), lambda i: (i, 0)),
            pl.BlockSpec((512, 32), lambda i: (0, 0)),
        ],
        out_specs=[
            pl.BlockSpec((_RB, 1), lambda i: (i, 0)),
            pl.BlockSpec((_RB, 32), lambda i: (i, 0)),
        ],
        out_shape=[
            jax.ShapeDtypeStruct((n, 1), jnp.int32),
            jax.ShapeDtypeStruct((n, 32), jnp.float32),
        ],
    )(flat, emb)
    return idx3.reshape(n), quant


def kernel(x, emb, w_c1, b_c1, w_c2, b_c2, w_c3, b_c3, w_c4, b_c4,
           w_d1, b_d1, w_d2, b_d2, w_d3, b_d3, w_d4, b_d4, w_d5, b_d5):
    h = jax.nn.relu(_conv2d(x, w_c1, b_c1, 2, 1))
    h = jax.nn.relu(_conv2d(h, w_c2, b_c2, 2, 1))
    h = jax.nn.relu(_conv2d(h, w_c3, b_c3, 2, 1))
    h = _conv2d(h, w_c4, b_c4, 1, 0)
    latents = jnp.transpose(h, (0, 2, 3, 1))
    flat = latents.reshape(-1, latents.shape[-1])

    idx, quantized = _vq(flat, emb)
    quantized = quantized.reshape(latents.shape)
    indices = idx.reshape(latents.shape[:-1])

    d = jnp.transpose(quantized, (0, 3, 1, 2))
    d = _deconv2d(d, w_d1, b_d1, 2, 1)
    d = _deconv2d(d, w_d2, b_d2, 2, 1)
    d = _deconv2d(d, w_d3, b_d3, 2, 1)
    d = _deconv2d(d, w_d4, b_d4, 2, 1)
    d = _deconv2d(d, w_d5, b_d5, 1, 1)
    d = _resize_bilinear_align_corners(d, 28, 28)
    x_recon = jax.nn.sigmoid(d)
    return x_recon, indices


# speedup vs baseline: 5.4180x; 5.4180x over previous
import numpy as np

import jax
import jax.numpy as jnp
from jax.experimental import pallas as pl


def _conv2d(x, w, b, stride, pad):
    out = jax.lax.conv_general_dilated(
        x, w, window_strides=(stride, stride), padding=((pad, pad), (pad, pad)),
        dimension_numbers=('NCHW', 'OIHW', 'NCHW'))
    return out + b[None, :, None, None]


_RB = 1568


def _vq_body(flat_ref, emb_ref, idx_ref, quant_ref):
    f = flat_ref[...]
    e = emb_ref[...]
    prod = jax.lax.dot_general(f, e, (((1,), (1,)), ((), ())),
                               preferred_element_type=jnp.float32)
    e2row = jax.lax.dot_general(
        jnp.ones((1, 32), jnp.float32), e * e, (((1,), (1,)), ((), ())),
        precision=jax.lax.Precision.HIGHEST,
        preferred_element_type=jnp.float32)
    scores = e2row - 2.0 * prod
    m = jnp.min(scores, axis=1, keepdims=True)
    iota = jax.lax.broadcasted_iota(jnp.int32, scores.shape, 1)
    idxv = jnp.min(jnp.where(scores == m, iota, 512), axis=1,
                   keepdims=True)
    idx_ref[...] = idxv
    onehot = (iota == idxv).astype(jnp.float32)
    quant_ref[...] = jax.lax.dot_general(
        onehot, e, (((1,), (0,)), ((), ())),
        precision=jax.lax.Precision.HIGHEST,
        preferred_element_type=jnp.float32)


def _vq(flat, emb):
    n = flat.shape[0]
    grid = n // _RB
    idx2, quant = pl.pallas_call(
        _vq_body,
        grid=(grid,),
        in_specs=[
            pl.BlockSpec((_RB, 32), lambda i: (i, 0)),
            pl.BlockSpec((512, 32), lambda i: (0, 0)),
        ],
        out_specs=[
            pl.BlockSpec((_RB, 1), lambda i: (i, 0)),
            pl.BlockSpec((_RB, 32), lambda i: (i, 0)),
        ],
        out_shape=[
            jax.ShapeDtypeStruct((n, 1), jnp.int32),
            jax.ShapeDtypeStruct((n, 32), jnp.float32),
        ],
    )(flat, emb)
    return idx2.reshape(n), quant


def _axis_windows():
    ys = np.linspace(0.0, 447.0, 28)
    y0 = np.floor(ys).astype(np.int64)
    wy = ys - y0
    By = np.stack([1.0 - wy, wy], axis=1).astype(np.float32)

    params = [
        (1, 3, 1, 448, 448),
        (2, 4, 1, 448, 224),
        (2, 4, 1, 224, 112),
        (2, 4, 1, 112, 56),
        (2, 4, 1, 56, 28),
    ]
    ry = y0.copy()
    w_out = 2
    stages = []
    for (s, k, p, s_out, s_in) in params:
        ry_in = -((-(ry + p - (k - 1))) // s)
        T = np.zeros((28, w_out, 4, k), np.float32)
        for nn in range(28):
            for di in range(w_out):
                o = ry[nn] + di
                if not (0 <= o < s_out):
                    continue
                for t in range(k):
                    num = o + p - t
                    if num % s:
                        continue
                    i = num // s
                    dii = i - ry_in[nn]
                    if 0 <= dii < 4 and 0 <= i < s_in:
                        T[nn, di, dii, t] = 1.0
        stages.append(T)
        ry = ry_in
        w_out = 4
    return y0, By, ry, stages


_Y0, _BY, _RY0, _STAGES = _axis_windows()


def _fused_operator(w_d1, b_d1, w_d2, b_d2, w_d3, b_d3, w_d4, b_d4, w_d5, b_d5):
    prec = 'highest'
    By = jnp.asarray(_BY)
    A = (By[:, None, :, None] * By[None, :, None, :])[:, :, None, :, :]
    bias = jnp.zeros((28, 28), jnp.float32)
    layer_ws = [(w_d5, b_d5), (w_d4, b_d4), (w_d3, b_d3), (w_d2, b_d2),
                (w_d1, b_d1)]
    for (T, (Wl, bl)) in zip(_STAGES, layer_ws):
        Tj = jnp.asarray(T)
        bias = bias + jnp.einsum('nmoab,o->nm', A, bl, precision=prec)
        ci, co = Wl.shape[0], Wl.shape[1]
        if ci < co:
            t1 = jnp.einsum('nmoab,iotu->nmiabtu', A, Wl, precision=prec)
            t2 = jnp.einsum('nmiabtu,naxt->nmixbu', t1, Tj, precision=prec)
            A = jnp.einsum('nmixbu,mbyu->nmixy', t2, Tj, precision=prec)
        else:
            t1 = jnp.einsum('nmoab,naxt->nmobxt', A, Tj, precision=prec)
            t2 = jnp.einsum('nmobxt,mbyu->nmoxytu', t1, Tj, precision=prec)
            A = jnp.einsum('nmoxytu,iotu->nmixy', t2, Wl, precision=prec)
    return A, bias


def _dec_body(qwin_ref, a_ref, bias_ref, out_ref):
    q = qwin_ref[0]
    m = a_ref[...] * q
    s = jax.lax.dot_general(
        m, jnp.ones((512, 1), jnp.float32), (((1,), (0,)), ((), ())),
        precision=jax.lax.Precision.HIGHEST,
        preferred_element_type=jnp.float32)
    out_ref[0] = jax.nn.sigmoid(s + bias_ref[...])


def _fused_decode(quantized_nhwc, A0, bias):
    B = quantized_nhwc.shape[0]
    qpad = jnp.pad(quantized_nhwc, ((0, 0), (2, 2), (2, 2), (0, 0)))
    rows4 = _RY0[:, None] + 2 + np.arange(4)[None, :]
    qr = qpad[:, rows4]
    qrc = qr[:, :, :, rows4]
    qwin = jnp.transpose(qrc, (0, 1, 3, 2, 4, 5)).reshape(B, 784, 512)
    A0r = jnp.transpose(A0, (0, 1, 3, 4, 2)).reshape(784, 512)
    biasc = bias.reshape(784, 1)
    out = pl.pallas_call(
        _dec_body,
        grid=(B,),
        in_specs=[
            pl.BlockSpec((1, 784, 512), lambda b: (b, 0, 0)),
            pl.BlockSpec((784, 512), lambda b: (0, 0)),
            pl.BlockSpec((784, 1), lambda b: (0, 0)),
        ],
        out_specs=pl.BlockSpec((1, 784, 1), lambda b: (b, 0, 0)),
        out_shape=jax.ShapeDtypeStruct((B, 784, 1), jnp.float32),
    )(qwin, A0r, biasc)
    return out.reshape(B, 1, 28, 28)


def kernel(x, emb, w_c1, b_c1, w_c2, b_c2, w_c3, b_c3, w_c4, b_c4,
           w_d1, b_d1, w_d2, b_d2, w_d3, b_d3, w_d4, b_d4, w_d5, b_d5):
    h = jax.nn.relu(_conv2d(x, w_c1, b_c1, 2, 1))
    h = jax.nn.relu(_conv2d(h, w_c2, b_c2, 2, 1))
    h = jax.nn.relu(_conv2d(h, w_c3, b_c3, 2, 1))
    h = _conv2d(h, w_c4, b_c4, 1, 0)
    latents = jnp.transpose(h, (0, 2, 3, 1))
    flat = latents.reshape(-1, latents.shape[-1])

    idx, quantized = _vq(flat, emb)
    indices = idx.reshape(latents.shape[:-1])
    quantized = quantized.reshape(latents.shape)

    A0, bias = _fused_operator(w_d1, b_d1, w_d2, b_d2, w_d3, b_d3,
                               w_d4, b_d4, w_d5, b_d5)
    x_recon = _fused_decode(quantized, A0, bias)
    return x_recon, indices


# speedup vs baseline: 5.7371x; 1.0589x over previous
import functools

import numpy as np

import jax
import jax.numpy as jnp
from jax.experimental import pallas as pl
from jax.experimental.pallas import tpu as pltpu
from jax.experimental.pallas import tpu_sc as plsc


def _conv2d(x, w, b, stride, pad):
    out = jax.lax.conv_general_dilated(
        x, w, window_strides=(stride, stride), padding=((pad, pad), (pad, pad)),
        dimension_numbers=('NCHW', 'OIHW', 'NCHW'))
    return out + b[None, :, None, None]


_RB = 1568


def _vq_body(flat_ref, emb_ref, idx_ref):
    f = flat_ref[...]
    e = emb_ref[...]
    prod = jax.lax.dot_general(f, e, (((1,), (1,)), ((), ())),
                               preferred_element_type=jnp.float32)
    e2row = jax.lax.dot_general(
        jnp.ones((1, 32), jnp.float32), e * e, (((1,), (1,)), ((), ())),
        precision=jax.lax.Precision.HIGHEST,
        preferred_element_type=jnp.float32)
    scores = e2row - 2.0 * prod
    m = jnp.min(scores, axis=1, keepdims=True)
    iota = jax.lax.broadcasted_iota(jnp.int32, scores.shape, 1)
    idxv = jnp.min(jnp.where(scores == m, iota, 512), axis=1,
                   keepdims=True)
    idx_ref[...] = idxv


def _vq_indices(flat, emb):
    n = flat.shape[0]
    grid = n // _RB
    idx2 = pl.pallas_call(
        _vq_body,
        grid=(grid,),
        in_specs=[
            pl.BlockSpec((_RB, 32), lambda i: (i, 0)),
            pl.BlockSpec((512, 32), lambda i: (0, 0)),
        ],
        out_specs=pl.BlockSpec((_RB, 1), lambda i: (i, 0)),
        out_shape=jax.ShapeDtypeStruct((n, 1), jnp.int32),
    )(flat, emb)
    return idx2.reshape(n)


@functools.lru_cache(maxsize=None)
def _make_sc_gather(B, V, D):
    info = plsc.get_sparse_core_info()
    NW = info.num_cores * info.num_subcores
    assert D % info.num_lanes == 0 and B % (8 * NW) == 0
    b_per_w = B // NW
    mesh = plsc.VectorSubcoreMesh(core_axis_name="c", subcore_axis_name="s")

    @functools.partial(
        pl.kernel, mesh=mesh,
        out_type=jax.ShapeDtypeStruct((B, D), jnp.float32),
        scratch_types=[
            pltpu.VMEM((b_per_w,), jnp.int32),
            pltpu.VMEM((b_per_w, D), jnp.float32),
            pltpu.SemaphoreType.DMA,
        ],
    )
    def gather_k(table_hbm, idx_hbm, out_hbm, idx_v, rows_v, sem):
        wid = jax.lax.axis_index("s") * info.num_cores + jax.lax.axis_index("c")
        base = wid * b_per_w
        pltpu.sync_copy(idx_hbm.at[pl.ds(base, b_per_w)], idx_v)
        pltpu.async_copy(table_hbm.at[idx_v], rows_v, sem).wait()
        pltpu.sync_copy(rows_v, out_hbm.at[pl.ds(base, b_per_w)])

    return gather_k


def _sc_gather(emb, idx):
    d = emb.shape[1]
    emb_pad = jnp.pad(emb, ((0, 0), (0, 128 - d)))
    gather_k = _make_sc_gather(idx.shape[0], emb.shape[0], 128)
    return gather_k(emb_pad, idx)[:, :d]


def _axis_windows():
    ys = np.linspace(0.0, 447.0, 28)
    y0 = np.floor(ys).astype(np.int64)
    wy = ys - y0
    By = np.stack([1.0 - wy, wy], axis=1).astype(np.float32)

    params = [
        (1, 3, 1, 448, 448),
        (2, 4, 1, 448, 224),
        (2, 4, 1, 224, 112),
        (2, 4, 1, 112, 56),
        (2, 4, 1, 56, 28),
    ]
    ry = y0.copy()
    w_out = 2
    stages = []
    for (s, k, p, s_out, s_in) in params:
        ry_in = -((-(ry + p - (k - 1))) // s)
        T = np.zeros((28, w_out, 4, k), np.float32)
        for nn in range(28):
            for di in range(w_out):
                o = ry[nn] + di
                if not (0 <= o < s_out):
                    continue
                for t in range(k):
                    num = o + p - t
                    if num % s:
                        continue
                    i = num // s
                    dii = i - ry_in[nn]
                    if 0 <= dii < 4 and 0 <= i < s_in:
                        T[nn, di, dii, t] = 1.0
        stages.append(T)
        ry = ry_in
        w_out = 4
    return y0, By, ry, stages


_Y0, _BY, _RY0, _STAGES = _axis_windows()


def _fused_operator(w_d1, b_d1, w_d2, b_d2, w_d3, b_d3, w_d4, b_d4, w_d5, b_d5):
    prec = 'highest'
    By = jnp.asarray(_BY)
    A = (By[:, None, :, None] * By[None, :, None, :])[:, :, None, :, :]
    bias = jnp.zeros((28, 28), jnp.float32)
    layer_ws = [(w_d5, b_d5), (w_d4, b_d4), (w_d3, b_d3), (w_d2, b_d2),
                (w_d1, b_d1)]
    for (T, (Wl, bl)) in zip(_STAGES, layer_ws):
        Tj = jnp.asarray(T)
        bias = bias + jnp.einsum('nmoab,o->nm', A, bl, precision=prec)
        ci, co = Wl.shape[0], Wl.shape[1]
        if ci < co:
            t1 = jnp.einsum('nmoab,iotu->nmiabtu', A, Wl, precision=prec)
            t2 = jnp.einsum('nmiabtu,naxt->nmixbu', t1, Tj, precision=prec)
            A = jnp.einsum('nmixbu,mbyu->nmixy', t2, Tj, precision=prec)
        else:
            t1 = jnp.einsum('nmoab,naxt->nmobxt', A, Tj, precision=prec)
            t2 = jnp.einsum('nmobxt,mbyu->nmoxytu', t1, Tj, precision=prec)
            A = jnp.einsum('nmoxytu,iotu->nmixy', t2, Wl, precision=prec)
    return A, bias


def _dec_body(qwin_ref, a_ref, bias_ref, out_ref):
    q = qwin_ref[0]
    m = a_ref[...] * q
    s = jax.lax.dot_general(
        m, jnp.ones((512, 1), jnp.float32), (((1,), (0,)), ((), ())),
        precision=jax.lax.Precision.HIGHEST,
        preferred_element_type=jnp.float32)
    out_ref[0] = jax.nn.sigmoid(s + bias_ref[...])


def _fused_decode(quantized_nhwc, A0, bias):
    B = quantized_nhwc.shape[0]
    qpad = jnp.pad(quantized_nhwc, ((0, 0), (2, 2), (2, 2), (0, 0)))
    rows4 = _RY0[:, None] + 2 + np.arange(4)[None, :]
    qr = qpad[:, rows4]
    qrc = qr[:, :, :, rows4]
    qwin = jnp.transpose(qrc, (0, 1, 3, 2, 4, 5)).reshape(B, 784, 512)
    A0r = jnp.transpose(A0, (0, 1, 3, 4, 2)).reshape(784, 512)
    biasc = bias.reshape(784, 1)
    out = pl.pallas_call(
        _dec_body,
        grid=(B,),
        in_specs=[
            pl.BlockSpec((1, 784, 512), lambda b: (b, 0, 0)),
            pl.BlockSpec((784, 512), lambda b: (0, 0)),
            pl.BlockSpec((784, 1), lambda b: (0, 0)),
        ],
        out_specs=pl.BlockSpec((1, 784, 1), lambda b: (b, 0, 0)),
        out_shape=jax.ShapeDtypeStruct((B, 784, 1), jnp.float32),
    )(qwin, A0r, biasc)
    return out.reshape(B, 1, 28, 28)


def kernel(x, emb, w_c1, b_c1, w_c2, b_c2, w_c3, b_c3, w_c4, b_c4,
           w_d1, b_d1, w_d2, b_d2, w_d3, b_d3, w_d4, b_d4, w_d5, b_d5):
    h = jax.nn.relu(_conv2d(x, w_c1, b_c1, 2, 1))
    h = jax.nn.relu(_conv2d(h, w_c2, b_c2, 2, 1))
    h = jax.nn.relu(_conv2d(h, w_c3, b_c3, 2, 1))
    h = _conv2d(h, w_c4, b_c4, 1, 0)
    latents = jnp.transpose(h, (0, 2, 3, 1))
    flat = latents.reshape(-1, latents.shape[-1])

    idx = _vq_indices(flat, emb)
    indices = idx.reshape(latents.shape[:-1])
    quantized = _sc_gather(emb, idx).reshape(latents.shape)

    A0, bias = _fused_operator(w_d1, b_d1, w_d2, b_d2, w_d3, b_d3,
                               w_d4, b_d4, w_d5, b_d5)
    x_recon = _fused_decode(quantized, A0, bias)
    return x_recon, indices
